# Initial kernel scaffold; baseline (speedup 1.0000x reference)
import jax
import jax.numpy as jnp
from jax.experimental import pallas as pl


def kernel(image, bkv, l1, g1, bm1, l2, g2, bm2, l3, g3, bm3, l4, g4, bm4, l5, g5, bm5, w1p, fb1, w2, fb2, w3, fb3):
    raise NotImplementedError("write your pallas kernel here")


# speedup vs baseline: 2.0794x; 2.0794x over previous
import jax
import jax.numpy as jnp
from jax.experimental import pallas as pl
from jax.experimental.pallas import tpu as pltpu

_G = 4
_CDT = jnp.float32

_VMEM = pl.BlockSpec(memory_space=pltpu.MemorySpace.VMEM)


def _fused_kernel(x_ref, bkv_ref, l1_ref, g1_ref, bm1_ref,
                  l2_ref, g2_ref, bm2_ref,
                  l3_ref, g3_ref, bm3_ref,
                  l4_ref, g4_ref, bm4_ref,
                  l5_ref, g5_ref, bm5_ref,
                  w1_ref, fb1_ref, w2_ref, fb2_ref, w3_ref, fb3_ref,
                  o_ref):
    f32 = jnp.float32

    def dot(a, b):
        return jnp.dot(a, b, preferred_element_type=f32)

    talls = []
    for i in range(_G):
        xi = x_ref[i]
        kv = dot(xi[0:252, :], bkv_ref[0])
        for di in range(1, 5):
            kv = kv + dot(xi[di:di + 252, :], bkv_ref[di])
        talls.append(dot(l1_ref[...], kv.astype(_CDT)).astype(_CDT))

    acc1 = jnp.concatenate([bm1_ref[...]] * _G, axis=0).astype(f32)
    for di in range(5):
        tcat = jnp.concatenate(
            [talls[i][di * 128:(di + 1) * 128, :] for i in range(_G)], axis=0)
        acc1 = acc1 + dot(tcat, g1_ref[di])

    x2 = acc1.astype(_CDT)
    acc2 = bm2_ref[...].astype(f32)
    for di in range(3):
        t2 = jnp.concatenate(
            [dot(l2_ref[di], x2[i * 128:(i + 1) * 128, :]).astype(_CDT)
             for i in range(_G)], axis=0)
        acc2 = acc2 + dot(t2, g2_ref[di])

    x3 = acc2.astype(_CDT)
    acc3 = bm3_ref[...].astype(f32)
    for di in range(3):
        t3 = dot(l3_ref[di], x3)
        acc3 = acc3 + dot(t3.astype(_CDT), g3_ref[di])

    x4 = acc3.astype(_CDT)
    acc4 = bm4_ref[...].astype(f32)
    for di in range(3):
        t4 = dot(l4_ref[di], x4)
        acc4 = acc4 + dot(t4.astype(_CDT), g4_ref[di])

    x5 = acc4.astype(_CDT)
    acc5 = bm5_ref[...].astype(f32)
    for di in range(5):
        t5 = dot(l5_ref[di], x5)
        acc5 = acc5 + dot(t5.astype(_CDT), g5_ref[di])

    a5 = acc5.astype(_CDT)
    h = fb1_ref[...]
    for r in range(4):
        h = h + dot(a5[r * _G:(r + 1) * _G, :], w1_ref[r])
    h = jnp.maximum(h, 0.0)
    h = jnp.maximum(dot(h.astype(_CDT), w2_ref[...]) + fb2_ref[...], 0.0)
    logits = dot(h.astype(_CDT), w3_ref[...]) + fb3_ref[...]
    m = jnp.max(logits, axis=-1, keepdims=True)
    lse = m + jnp.log(jnp.sum(jnp.exp(logits - m), axis=-1, keepdims=True))
    o_ref[...] = logits - lse


def _blockdiag(lm, rp, cp):
    k, r, c = lm.shape
    lmp = jnp.pad(lm, ((0, 0), (0, rp - r), (0, cp - c)))
    eye = jnp.eye(_G, dtype=lm.dtype)
    return jnp.einsum('ij,krc->kirjc', eye, lmp).reshape(k, _G * rp, _G * cp)


def _tile_rows(bm, rp):
    return jnp.tile(jnp.pad(bm, ((0, rp - bm.shape[0]), (0, 0))), (_G, 1))


def kernel(image, bkv, l1, g1, bm1, l2, g2, bm2, l3, g3, bm3,
           l4, g4, bm4, l5, g5, bm5, w1p, fb1, w2, fb2, w3, fb3):
    n = image.shape[0]
    assert n % _G == 0
    bf = _CDT

    x = image.reshape(n, 256, 256).astype(bf)
    bkvp = jnp.pad(bkv, ((0, 0), (0, 0), (0, 4))).astype(bf)
    l1a = jnp.pad(l1, ((0, 0), (0, 4), (0, 0))).reshape(640, 252).astype(bf)
    g1p = jnp.pad(g1, ((0, 0), (0, 4), (0, 0))).astype(bf)
    bm1g = jnp.pad(bm1, ((0, 4), (0, 0))).astype(bf)

    l2p = jnp.pad(l2, ((0, 0), (0, 3), (0, 4))).astype(bf)
    g2b = g2.astype(bf)
    bm2g = _tile_rows(bm2, 64).astype(bf)

    l3b = _blockdiag(l3, 32, 64).astype(bf)
    g3b = g3.astype(bf)
    bm3g = _tile_rows(bm3, 32).astype(bf)

    l4b = _blockdiag(l4, 16, 32).astype(bf)
    g4b = g4.astype(bf)
    bm4g = _tile_rows(bm4, 16).astype(bf)

    l5pad = jnp.pad(l5, ((0, 0), (0, 0), (0, 3)))
    eyeg = jnp.eye(_G, dtype=l5.dtype)
    l5b = jnp.einsum('ij,krc->krijc', eyeg, l5pad).reshape(
        5, 4 * _G, 16 * _G).astype(bf)
    g5b = g5.astype(bf)
    bm5g = jnp.repeat(bm5, _G, axis=0).astype(bf)

    w1f = w1p.astype(bf)
    w2b = w2.astype(bf)
    w3b = w3.astype(bf)

    out = pl.pallas_call(
        _fused_kernel,
        out_shape=jax.ShapeDtypeStruct((n // _G, _G, 2), jnp.float32),
        grid=(n // _G,),
        in_specs=[pl.BlockSpec((_G, 256, 256), lambda i: (i, 0, 0))]
                 + [_VMEM] * 22,
        out_specs=pl.BlockSpec((None, _G, 2), lambda i: (i, 0, 0)),
        compiler_params=pltpu.CompilerParams(
            dimension_semantics=("parallel",),
            vmem_limit_bytes=62 * 1024 * 1024),
    )(x, bkvp, l1a, g1p, bm1g, l2p, g2b, bm2g, l3b, g3b, bm3g,
      l4b, g4b, bm4g, l5b, g5b, bm5g, w1f, fb1, w2b, fb2, w3b, fb3)
    return out.reshape(n, 2)
